# baseline (speedup 1.0000x reference)
import jax
import jax.numpy as jnp
import numpy as np
from jax.experimental import pallas as pl
from jax.experimental.pallas import tpu as pltpu

_B = 8192
_D = 128
_T = 8
_K = 2
_L = 3
_NF = 8
_VR = 16
_NP = _VR * _VR
_BB = 2048


def _gelu(x):
    return x * 0.5 * (1.0 + jax.lax.erf(x * np.float32(1.0 / np.sqrt(2.0))))


def _ln(x, g, b):
    m = jnp.mean(x, axis=-1, keepdims=True)
    xc = x - m
    v = jnp.mean(xc * xc, axis=-1, keepdims=True)
    return xc * jax.lax.rsqrt(v + 1e-5) * g + b


def _net_kernel(a_ref, b_ref, Wi_ref, bi_ref, lig_ref, lib_ref,
                Wr_ref, br_ref, W1_ref, b1_ref, W2s_ref, b2_ref,
                lng_ref, lnb_ref, Ws1_ref, bs1_ref, Ws2_ref, bs2_ref,
                Wd1_ref, bd1_ref, Wd2_ref, bd2_ref,
                sum_ref, diff_ref, rout_ref, aux_ref,
                tab_ref, ps_ref, oh_ref, cnt_ref):
    i = pl.program_id(0)
    nsteps = pl.num_programs(0)

    @pl.when(i == 0)
    def _build_tables():
        cnt_ref[...] = jnp.zeros_like(cnt_ref)
        pair = jax.lax.broadcasted_iota(jnp.int32, (_NP, 1), 0)
        pa = (pair // _VR).astype(jnp.float32)
        pb = (pair % _VR).astype(jnp.float32)
        fexp = jax.lax.broadcasted_iota(jnp.int32, (1, _NF), 1)
        freqs = jnp.exp2(fexp.astype(jnp.float32)) * np.float32(
            2.0 * np.pi / _VR)
        av = pa * freqs
        bv = pb * freqs
        x0 = jnp.concatenate(
            [jnp.sin(av), jnp.cos(av), jnp.sin(bv), jnp.cos(bv)], axis=1)

        x = jnp.dot(x0, Wi_ref[...], preferred_element_type=jnp.float32)
        x = _gelu(_ln(x + bi_ref[0], lig_ref[0], lib_ref[0]))

        ti = jax.lax.broadcasted_iota(jnp.int32, (_NP, _T), 1)
        i1 = i2 = None
        for l in range(_L):
            logits = jnp.dot(x, Wr_ref[l], preferred_element_type=jnp.float32)
            logits = logits + br_ref[l]
            p = logits - jnp.max(logits, axis=1, keepdims=True)
            p = jnp.exp(p)
            p = p / jnp.sum(p, axis=1, keepdims=True)

            m1 = jnp.max(p, axis=1, keepdims=True)
            i1 = jnp.min(jnp.where(p == m1, ti, _T), axis=1, keepdims=True)
            p_rest = jnp.where(ti == i1, -jnp.inf, p)
            m2 = jnp.max(p_rest, axis=1, keepdims=True)
            i2 = jnp.min(jnp.where(p_rest == m2, ti, _T), axis=1,
                         keepdims=True)
            denom = 1.0 / (m1 + m2 + 1e-9)
            oh = ((ti == i1) | (ti == i2)).astype(jnp.float32)
            gates_full = jnp.where(ti == i1, m1 * denom, 0.0) + jnp.where(
                ti == i2, m2 * denom, 0.0)

            ps_ref[:, l * _T:(l + 1) * _T] = p
            oh_ref[:, l * _T:(l + 1) * _T] = oh

            h_all = jnp.concatenate(
                [jnp.dot(x, W1_ref[l, t], preferred_element_type=jnp.float32)
                 for t in range(_T)], axis=1)
            h_all = _gelu(h_all + b1_ref[l])
            gh = jnp.concatenate(
                [h_all[:, t * _D:(t + 1) * _D] * gates_full[:, t][:, None]
                 for t in range(_T)], axis=1)
            out = jnp.dot(gh, W2s_ref[l], preferred_element_type=jnp.float32)
            out = out + jnp.dot(gates_full, b2_ref[l],
                                preferred_element_type=jnp.float32)
            x = _ln(x + out, lng_ref[l], lnb_ref[l])

        sl = jnp.dot(_gelu(jnp.dot(x, Ws1_ref[...],
                                   preferred_element_type=jnp.float32)
                           + bs1_ref[0]),
                     Ws2_ref[...], preferred_element_type=jnp.float32)
        dl = jnp.dot(_gelu(jnp.dot(x, Wd1_ref[...],
                                   preferred_element_type=jnp.float32)
                           + bd1_ref[0]),
                     Wd2_ref[...], preferred_element_type=jnp.float32)
        tab = jnp.concatenate(
            [sl + bs2_ref[0], dl + bd2_ref[0],
             i1.astype(jnp.float32), i2.astype(jnp.float32),
             jnp.zeros((_NP, _D - 13), jnp.float32)], axis=1)
        tab_ref[...] = tab

    pair_t = (a_ref[0, 0, :] * _VR + b_ref[0, 0, :])[:, None]
    onehot = (pair_t == jax.lax.broadcasted_iota(
        jnp.int32, (_BB, _NP), 1)).astype(jnp.float32)
    g = jnp.dot(onehot, tab_ref[...], preferred_element_type=jnp.float32)
    sum_ref[...] = g[:, 0:5]
    diff_ref[...] = g[:, 5:11]
    rout_ref[...] = g[:, 11:13].astype(jnp.int32)
    cnt_ref[...] += jnp.sum(onehot, axis=0, keepdims=True)

    @pl.when(i == nsteps - 1)
    def _fin():
        ps_sum = jnp.dot(cnt_ref[...], ps_ref[...],
                         preferred_element_type=jnp.float32)
        ls_sum = jnp.dot(cnt_ref[...], oh_ref[...],
                         preferred_element_type=jnp.float32)
        scale = np.float32(_T) / np.float32(_B * _B)
        aux_ref[...] = jnp.sum(ps_sum * ls_sum, keepdims=True).reshape(
            1, 1) * scale


@jax.jit
def _run(a, b, params):
    nb = _B // _BB
    a3 = a.astype(jnp.int32).reshape(nb, 1, _BB)
    b3 = b.astype(jnp.int32).reshape(nb, 1, _BB)
    p = params
    row = lambda v: v.reshape(1, -1)

    full = lambda s: pl.BlockSpec(s, lambda i: (0,) * len(s))
    in_specs = [
        pl.BlockSpec((1, 1, _BB), lambda i: (i, 0, 0)),
        pl.BlockSpec((1, 1, _BB), lambda i: (i, 0, 0)),
        full((4 * _NF, _D)), full((1, _D)), full((1, _D)), full((1, _D)),
        full((_L, _D, _T)), full((_L, _T)),
        full((_L, _T, _D, _D)), full((_L, _T * _D)),
        full((_L, _T * _D, _D)), full((_L, _T, _D)),
        full((_L, _D)), full((_L, _D)),
        full((_D, _D // 2)), full((1, _D // 2)),
        full((_D // 2, 5)), full((1, 5)),
        full((_D, _D // 2)), full((1, _D // 2)),
        full((_D // 2, 6)), full((1, 6)),
    ]
    out_specs = [
        pl.BlockSpec((_BB, 5), lambda i: (i, 0)),
        pl.BlockSpec((_BB, 6), lambda i: (i, 0)),
        pl.BlockSpec((_BB, _K), lambda i: (i, 0)),
        pl.BlockSpec((1, 1), lambda i: (0, 0)),
    ]
    out_shape = [
        jax.ShapeDtypeStruct((_B, 5), jnp.float32),
        jax.ShapeDtypeStruct((_B, 6), jnp.float32),
        jax.ShapeDtypeStruct((_B, _K), jnp.int32),
        jax.ShapeDtypeStruct((1, 1), jnp.float32),
    ]
    sl, dl, rout, aux = pl.pallas_call(
        _net_kernel,
        grid=(nb,),
        in_specs=in_specs,
        out_specs=out_specs,
        out_shape=out_shape,
        scratch_shapes=[
            pltpu.VMEM((_NP, _D), jnp.float32),
            pltpu.VMEM((_NP, _L * _T), jnp.float32),
            pltpu.VMEM((_NP, _L * _T), jnp.float32),
            pltpu.VMEM((1, _NP), jnp.float32),
        ],
    )(a3, b3,
      p["Wi"], row(p["bi"]), row(p["ln_in_g"]), row(p["ln_in_b"]),
      p["Wr"], p["br"],
      p["W1"], p["b1"].reshape(_L, _T * _D),
      p["W2"].reshape(_L, _T * _D, _D), p["b2"],
      p["ln_g"], p["ln_b"],
      p["Ws1"], row(p["bs1"]), p["Ws2"], row(p["bs2"]),
      p["Wd1"], row(p["bd1"]), p["Wd2"], row(p["bd2"]))
    return sl, dl, rout, aux[0, 0]


def kernel(a, b, params):
    return _run(a, b, params)


# speedup vs baseline: 1.0582x; 1.0582x over previous
import jax
import jax.numpy as jnp
import numpy as np
from jax.experimental import pallas as pl
from jax.experimental.pallas import tpu as pltpu

_B = 8192
_D = 128
_T = 8
_K = 2
_L = 3
_NF = 8
_VR = 16
_NP = _VR * _VR
_BB = 2048


def _gelu(x):
    return x * 0.5 * (1.0 + jax.lax.erf(x * np.float32(1.0 / np.sqrt(2.0))))


def _ln(x, g, b):
    m = jnp.mean(x, axis=-1, keepdims=True)
    xc = x - m
    v = jnp.mean(xc * xc, axis=-1, keepdims=True)
    return xc * jax.lax.rsqrt(v + 1e-5) * g + b


def _net_kernel(a_ref, b_ref, Wi_ref, bi_ref, lig_ref, lib_ref,
                Wr_ref, br_ref, W1_ref, b1_ref, W2_ref, b2_ref,
                lng_ref, lnb_ref, Ws1_ref, bs1_ref, Ws2_ref, bs2_ref,
                Wd1_ref, bd1_ref, Wd2_ref, bd2_ref,
                sum_ref, diff_ref, rout_ref, aux_ref,
                tab_ref, ps_ref, oh_ref, cnt_ref):
    i = pl.program_id(0)
    nsteps = pl.num_programs(0)

    @pl.when(i == 0)
    def _build_tables():
        cnt_ref[...] = jnp.zeros_like(cnt_ref)
        pair = jax.lax.broadcasted_iota(jnp.int32, (_NP, 1), 0)
        pa = (pair // _VR).astype(jnp.float32)
        pb = (pair % _VR).astype(jnp.float32)
        fexp = jax.lax.broadcasted_iota(jnp.int32, (1, _NF), 1)
        freqs = jnp.exp2(fexp.astype(jnp.float32)) * np.float32(
            2.0 * np.pi / _VR)
        av = pa * freqs
        bv = pb * freqs
        x0 = jnp.concatenate(
            [jnp.sin(av), jnp.cos(av), jnp.sin(bv), jnp.cos(bv)], axis=1)

        x = jnp.dot(x0, Wi_ref[...], preferred_element_type=jnp.float32)
        x = _gelu(_ln(x + bi_ref[...], lig_ref[...], lib_ref[...]))

        ti = jax.lax.broadcasted_iota(jnp.int32, (_NP, _T), 1)
        i1 = i2 = None
        for l in range(_L):
            logits = jnp.dot(x, Wr_ref[l], preferred_element_type=jnp.float32)
            logits = logits + br_ref[l]
            p = logits - jnp.max(logits, axis=1, keepdims=True)
            p = jnp.exp(p)
            p = p / jnp.sum(p, axis=1, keepdims=True)

            m1 = jnp.max(p, axis=1, keepdims=True)
            i1 = jnp.min(jnp.where(p == m1, ti, _T), axis=1, keepdims=True)
            p_rest = jnp.where(ti == i1, -jnp.inf, p)
            m2 = jnp.max(p_rest, axis=1, keepdims=True)
            i2 = jnp.min(jnp.where(p_rest == m2, ti, _T), axis=1,
                         keepdims=True)
            denom = 1.0 / (m1 + m2 + 1e-9)
            oh = ((ti == i1) | (ti == i2)).astype(jnp.float32)
            gates_full = jnp.where(ti == i1, m1 * denom, 0.0) + jnp.where(
                ti == i2, m2 * denom, 0.0)

            ps_ref[:, l * _T:(l + 1) * _T] = p
            oh_ref[:, l * _T:(l + 1) * _T] = oh

            out = jnp.zeros((_NP, _D), jnp.float32)
            for t in range(_T):
                h = jnp.dot(x, W1_ref[l, t],
                            preferred_element_type=jnp.float32)
                h = _gelu(h + b1_ref[l, t])
                eo = jnp.dot(h, W2_ref[l, t],
                             preferred_element_type=jnp.float32)
                eo = eo + b2_ref[l, t]
                out = out + gates_full[:, t][:, None] * eo
            x = _ln(x + out, lng_ref[l], lnb_ref[l])

        sl = jnp.dot(_gelu(jnp.dot(x, Ws1_ref[...],
                                   preferred_element_type=jnp.float32)
                           + bs1_ref[...]),
                     Ws2_ref[...], preferred_element_type=jnp.float32)
        dl = jnp.dot(_gelu(jnp.dot(x, Wd1_ref[...],
                                   preferred_element_type=jnp.float32)
                           + bd1_ref[...]),
                     Wd2_ref[...], preferred_element_type=jnp.float32)
        tab = jnp.concatenate(
            [sl + bs2_ref[...], dl + bd2_ref[...],
             i1.astype(jnp.float32), i2.astype(jnp.float32),
             jnp.zeros((_NP, _D - 13), jnp.float32)], axis=1)
        tab_ref[...] = tab

    pair_t = (a_ref[...] * _VR + b_ref[...])[:, None]
    onehot = (pair_t == jax.lax.broadcasted_iota(
        jnp.int32, (_BB, _NP), 1)).astype(jnp.float32)
    g = jnp.dot(onehot, tab_ref[...], preferred_element_type=jnp.float32)
    sum_ref[...] = g[:, 0:5]
    diff_ref[...] = g[:, 5:11]
    rout_ref[...] = g[:, 11:13].astype(jnp.int32)
    cnt_ref[...] += jnp.sum(onehot, axis=0, keepdims=True)

    @pl.when(i == nsteps - 1)
    def _fin():
        ps_sum = jnp.dot(cnt_ref[...], ps_ref[...],
                         preferred_element_type=jnp.float32)
        ls_sum = jnp.dot(cnt_ref[...], oh_ref[...],
                         preferred_element_type=jnp.float32)
        scale = np.float32(_T) / np.float32(_B * _B)
        aux_ref[...] = jnp.sum(ps_sum * ls_sum, keepdims=True).reshape(
            1, 1) * scale


@jax.jit
def _run(a, b, params):
    nb = _B // _BB
    p = params

    full = lambda s: pl.BlockSpec(s, lambda i: (0,) * len(s))
    in_specs = [
        pl.BlockSpec((_BB,), lambda i: (i,)),
        pl.BlockSpec((_BB,), lambda i: (i,)),
        full((4 * _NF, _D)), full((_D,)), full((_D,)), full((_D,)),
        full((_L, _D, _T)), full((_L, _T)),
        full((_L, _T, _D, _D)), full((_L, _T, _D)),
        full((_L, _T, _D, _D)), full((_L, _T, _D)),
        full((_L, _D)), full((_L, _D)),
        full((_D, _D // 2)), full((_D // 2,)),
        full((_D // 2, 5)), full((5,)),
        full((_D, _D // 2)), full((_D // 2,)),
        full((_D // 2, 6)), full((6,)),
    ]
    out_specs = [
        pl.BlockSpec((_BB, 5), lambda i: (i, 0)),
        pl.BlockSpec((_BB, 6), lambda i: (i, 0)),
        pl.BlockSpec((_BB, _K), lambda i: (i, 0)),
        pl.BlockSpec((1, 1), lambda i: (0, 0)),
    ]
    out_shape = [
        jax.ShapeDtypeStruct((_B, 5), jnp.float32),
        jax.ShapeDtypeStruct((_B, 6), jnp.float32),
        jax.ShapeDtypeStruct((_B, _K), jnp.int32),
        jax.ShapeDtypeStruct((1, 1), jnp.float32),
    ]
    sl, dl, rout, aux = pl.pallas_call(
        _net_kernel,
        grid=(nb,),
        in_specs=in_specs,
        out_specs=out_specs,
        out_shape=out_shape,
        scratch_shapes=[
            pltpu.VMEM((_NP, _D), jnp.float32),
            pltpu.VMEM((_NP, _L * _T), jnp.float32),
            pltpu.VMEM((_NP, _L * _T), jnp.float32),
            pltpu.VMEM((1, _NP), jnp.float32),
        ],
    )(a.astype(jnp.int32), b.astype(jnp.int32),
      p["Wi"], p["bi"], p["ln_in_g"], p["ln_in_b"],
      p["Wr"], p["br"], p["W1"], p["b1"], p["W2"], p["b2"],
      p["ln_g"], p["ln_b"],
      p["Ws1"], p["bs1"], p["Ws2"], p["bs2"],
      p["Wd1"], p["bd1"], p["Wd2"], p["bd2"])
    return sl, dl, rout, aux[0, 0]


def kernel(a, b, params):
    return _run(a, b, params)
